# baseline (speedup 1.0000x reference)
import functools

import jax
import jax.numpy as jnp
from jax.experimental import pallas as pl
from jax.experimental.pallas import tpu as pltpu
from jax.experimental.pallas import tpu_sc as plsc

N = 10000
E = 320000
G = 64
D_IN = 128
D_HID = 512
D_OUT = 64
N_CLASSES = 2

NC = 2
NS = 16
CHUNK = 64
NP = 10752
PADROWS = 512
EPAD = 327680
NCHUNKS = EPAD // CHUNK
CH_PER_CORE = NCHUNKS // NC
CH_PER_SUB = CH_PER_CORE // NS
ROWS_PER_SUB = NP // NS

R = 1344
NBLK = NP // R


@functools.lru_cache(maxsize=None)
def _make_propagate(D, gather=True):
    mesh = plsc.VectorSubcoreMesh(core_axis_name="c", subcore_axis_name="s")
    NBUF = 5
    IDXG = 20 if D > 64 else 160
    csub = EPAD // (NC * NS * CHUNK)
    ccore = csub * NS
    IDXG = min(IDXG, csub)
    NGRP = csub // IDXG

    @functools.partial(
        pl.kernel,
        mesh=mesh,
        out_type=jax.ShapeDtypeStruct((NC, NP, D), jnp.float32),
        scratch_types=[
            pltpu.VMEM_SHARED((NP, D), jnp.float32),
            pltpu.VMEM((IDXG, CHUNK), jnp.int32),
            pltpu.VMEM((IDXG, CHUNK), jnp.int32),
        ]
        + [pltpu.VMEM((CHUNK, D), jnp.float32) for _ in range(NBUF)]
        + [pltpu.SemaphoreType.DMA for _ in range(2 * NBUF)],
        compiler_params=pltpu.CompilerParams(use_tc_tiling_on_sc=False),
    )
    def prop(xs_hbm, row_hbm, col_hbm, zeros_hbm, out_hbm, acc, row_v, col_v,
             *bufs_and_sems):
        msg = bufs_and_sems[:NBUF]
        gsem = bufs_and_sems[NBUF:2 * NBUF]
        ssem = bufs_and_sems[2 * NBUF:]
        c = jax.lax.axis_index("c")
        s = jax.lax.axis_index("s")
        r0 = s * ROWS_PER_SUB
        base = c * ccore + s * csub
        pltpu.sync_copy(zeros_hbm.at[pl.ds(r0, ROWS_PER_SUB)],
                        acc.at[pl.ds(r0, ROWS_PER_SUB)])
        if not gather:
            pltpu.sync_copy(xs_hbm, msg[0])
        plsc.subcore_barrier()

        @pl.loop(0, NGRP)
        def _(grp):
            gbase = base + grp * IDXG
            pltpu.sync_copy(col_hbm.at[pl.ds(gbase, IDXG)], col_v)
            if gather:
                pltpu.sync_copy(row_hbm.at[pl.ds(gbase, IDXG)], row_v)
                for b in range(NBUF):
                    pltpu.async_copy(xs_hbm.at[row_v.at[b]], msg[b], gsem[b])

                @pl.loop(0, IDXG - NBUF, step=NBUF)
                def _(k):
                    for b in range(NBUF):
                        pltpu.make_async_copy(
                            xs_hbm.at[row_v.at[k + b]], msg[b], gsem[b]).wait()
                        pltpu.async_copy(msg[b], acc.at[col_v.at[k + b]],
                                         ssem[b], add=True)
                    for b in range(NBUF):
                        pltpu.make_async_copy(
                            msg[b], acc.at[col_v.at[k + b]], ssem[b]).wait()
                        pltpu.async_copy(
                            xs_hbm.at[row_v.at[k + NBUF + b]], msg[b], gsem[b])

                k0 = IDXG - NBUF
                for b in range(NBUF):
                    pltpu.make_async_copy(
                        xs_hbm.at[row_v.at[k0 + b]], msg[b], gsem[b]).wait()
                    pltpu.sync_copy(msg[b], acc.at[col_v.at[k0 + b]], add=True)
            else:
                @pl.loop(0, IDXG, step=NBUF)
                def _(k):
                    for b in range(NBUF):
                        pltpu.async_copy(msg[0], acc.at[col_v.at[k + b]],
                                         ssem[b], add=True)
                    for b in range(NBUF):
                        pltpu.make_async_copy(
                            msg[0], acc.at[col_v.at[k + b]], ssem[b]).wait()

        plsc.subcore_barrier()
        pltpu.sync_copy(acc.at[pl.ds(r0, ROWS_PER_SUB)],
                        out_hbm.at[c, pl.ds(r0, ROWS_PER_SUB)])

    return prop


def _tcB_body(deg_ref, x_ref, xs1_ref, dinvb_ref):
    cnt = deg_ref[0, :, 0:1] + deg_ref[1, :, 0:1] + 1.0
    dinv = jax.lax.rsqrt(cnt)
    dinvb = jnp.broadcast_to(dinv, (NP, D_IN))
    dinvb_ref[...] = dinvb
    xs1_ref[0:N, :] = dinvb[0:N, :] * x_ref[...]
    xs1_ref[N:NP, :] = jnp.zeros((NP - N, D_IN), jnp.float32)


def _tcB(deg_part, x):
    return pl.pallas_call(
        _tcB_body,
        out_shape=(
            jax.ShapeDtypeStruct((NP, D_IN), jnp.float32),
            jax.ShapeDtypeStruct((NP, D_IN), jnp.float32),
        ),
    )(deg_part, x)


def _tcD_body(s_ref, xs1_ref, dv_ref, W1_ref, b1_ref, W2_ref, xs2_ref):
    i = pl.program_id(0)
    dv = dv_ref[...]
    y1 = dv * (s_ref[0] + s_ref[1] + xs1_ref[...])
    h1 = jnp.maximum(jnp.dot(y1, W1_ref[...],
                             preferred_element_type=jnp.float32) + b1_ref[...], 0.0)
    t = jnp.dot(h1, W2_ref[...], preferred_element_type=jnp.float32)
    rowid = i * R + jax.lax.broadcasted_iota(jnp.int32, (R, 1), 0)
    xs2_ref[...] = jnp.where(rowid < N, dv[:, :D_OUT] * t, 0.0)


def _tcD(s1, xs1, dinvb, W1, b1, W2):
    return pl.pallas_call(
        _tcD_body,
        grid=(NBLK,),
        in_specs=[
            pl.BlockSpec((NC, R, D_IN), lambda i: (0, i, 0)),
            pl.BlockSpec((R, D_IN), lambda i: (i, 0)),
            pl.BlockSpec((R, D_IN), lambda i: (i, 0)),
            pl.BlockSpec((D_IN, D_HID), lambda i: (0, 0)),
            pl.BlockSpec((1, D_HID), lambda i: (0, 0)),
            pl.BlockSpec((D_HID, D_OUT), lambda i: (0, 0)),
        ],
        out_specs=pl.BlockSpec((R, D_OUT), lambda i: (i, 0)),
        out_shape=jax.ShapeDtypeStruct((NP, D_OUT), jnp.float32),
    )(s1, xs1, dinvb, W1, b1.reshape(1, D_HID), W2)


def _tcF_body(s2_ref, xs2_ref, dv_ref, b2_ref, batch_ref, Wf_ref, bf_ref,
              out_ref, S_acc):
    i = pl.program_id(0)

    @pl.when(i == 0)
    def _():
        S_acc[...] = jnp.zeros((G, 2 * D_OUT), jnp.float32)

    dv = dv_ref[:, :D_OUT]
    h2 = jnp.maximum(
        dv * (s2_ref[0] + s2_ref[1] + xs2_ref[...]) + b2_ref[...], 0.0)
    b = batch_ref[0, 0, :]
    onehot = (b[:, None] == jax.lax.broadcasted_iota(
        jnp.int32, (1, G), 1)).astype(jnp.float32)
    hext = jnp.concatenate([h2, jnp.ones((R, D_OUT), jnp.float32)], axis=1)
    S_acc[...] += jax.lax.dot_general(
        onehot, hext, (((0,), (0,)), ((), ())),
        preferred_element_type=jnp.float32)

    @pl.when(i == NBLK - 1)
    def _():
        S = S_acc[...]
        pooled = S[:, :D_OUT] / jnp.maximum(S[:, D_OUT:D_OUT + 1], 1.0)
        out_ref[...] = jnp.dot(pooled, Wf_ref[...],
                               preferred_element_type=jnp.float32) + bf_ref[...]


def _tcF(s2, xs2, dinvb, b2, batch3d, Wf, bf):
    return pl.pallas_call(
        _tcF_body,
        grid=(NBLK,),
        in_specs=[
            pl.BlockSpec((NC, R, D_OUT), lambda i: (0, i, 0)),
            pl.BlockSpec((R, D_OUT), lambda i: (i, 0)),
            pl.BlockSpec((R, D_IN), lambda i: (i, 0)),
            pl.BlockSpec((1, D_OUT), lambda i: (0, 0)),
            pl.BlockSpec((1, 1, R), lambda i: (i, 0, 0)),
            pl.BlockSpec((D_OUT, N_CLASSES), lambda i: (0, 0)),
            pl.BlockSpec((1, N_CLASSES), lambda i: (0, 0)),
        ],
        out_specs=pl.BlockSpec((G, N_CLASSES), lambda i: (0, 0)),
        out_shape=jax.ShapeDtypeStruct((G, N_CLASSES), jnp.float32),
        scratch_shapes=[pltpu.VMEM((G, 2 * D_OUT), jnp.float32)],
    )(s2, xs2, dinvb, b2.reshape(1, D_OUT), batch3d, Wf, bf.reshape(1, N_CLASSES))


def kernel(x, edge_index, batch, W1, b1, W2, b2, Wf, bf):
    pad = N + (jnp.arange(EPAD - E, dtype=jnp.int32) % PADROWS)
    row_m = jnp.concatenate([edge_index[0], pad]).reshape(NCHUNKS, CHUNK)
    col_m = jnp.concatenate([edge_index[1], pad]).reshape(NCHUNKS, CHUNK)
    ones16 = jnp.ones((CHUNK, 16), jnp.float32)
    z16 = jnp.zeros((NP, 16), jnp.float32)
    z128 = jnp.zeros((NP, D_IN), jnp.float32)
    z64 = jnp.zeros((NP, D_OUT), jnp.float32)
    batch3d = jnp.pad(batch, (0, NP - N), constant_values=G).reshape(NBLK, 1, R)

    deg_part = _make_propagate(16, gather=False)(
        ones16, row_m, col_m, z16)
    xs1, dinvb = _tcB(deg_part, x)
    s1 = _make_propagate(128)(xs1, row_m, col_m, z128)
    xs2 = _tcD(s1, xs1, dinvb, W1, b1, W2)
    s2 = _make_propagate(64)(xs2, row_m, col_m, z64)
    return _tcF(s2, xs2, dinvb, b2, batch3d, Wf, bf)


# speedup vs baseline: 1.0153x; 1.0153x over previous
import functools

import jax
import jax.numpy as jnp
from jax.experimental import pallas as pl
from jax.experimental.pallas import tpu as pltpu
from jax.experimental.pallas import tpu_sc as plsc

N = 10000
E = 320000
G = 64
D_IN = 128
D_HID = 512
D_OUT = 64
N_CLASSES = 2

NC = 2
NS = 16
CHUNK = 64
NP = 10752
PADROWS = 512
EPAD = 327680
NCHUNKS = EPAD // CHUNK
NREAL_CH = E // CHUNK
PAD_CH = (EPAD - E) // CHUNK
CH_PER_CORE = NCHUNKS // NC
CH_PER_SUB = CH_PER_CORE // NS
ROWS_PER_SUB = NP // NS

R = 1344
NBLK = NP // R


@functools.lru_cache(maxsize=None)
def _make_propagate(D, gather=True):
    mesh = plsc.VectorSubcoreMesh(core_axis_name="c", subcore_axis_name="s")
    NBUF = 4
    IDXG = 40
    csub = EPAD // (NC * NS * CHUNK)
    ccore = csub * NS
    NGRP = csub // IDXG

    @functools.partial(
        pl.kernel,
        mesh=mesh,
        out_type=jax.ShapeDtypeStruct((NC, NP, D), jnp.float32),
        scratch_types=[
            pltpu.VMEM_SHARED((NP, D), jnp.float32),
            pltpu.VMEM((IDXG, CHUNK), jnp.int32),
            pltpu.VMEM((IDXG, CHUNK), jnp.int32),
        ]
        + [pltpu.VMEM((CHUNK, D), jnp.float32) for _ in range(NBUF)]
        + [pltpu.SemaphoreType.DMA for _ in range(2 * NBUF)],
        compiler_params=pltpu.CompilerParams(use_tc_tiling_on_sc=False),
    )
    def prop(xs_hbm, ei_hbm, pad_hbm, zeros_hbm, out_hbm, acc, row_v, col_v,
             *bufs_and_sems):
        msg = bufs_and_sems[:NBUF]
        gsem = bufs_and_sems[NBUF:2 * NBUF]
        ssem = bufs_and_sems[2 * NBUF:]
        c = jax.lax.axis_index("c")
        s = jax.lax.axis_index("s")
        r0 = s * ROWS_PER_SUB
        base = c * ccore + s * csub
        pltpu.sync_copy(zeros_hbm.at[pl.ds(r0, ROWS_PER_SUB)],
                        acc.at[pl.ds(r0, ROWS_PER_SUB)])
        if not gather:
            pltpu.sync_copy(xs_hbm, msg[0])
        plsc.subcore_barrier()

        @pl.loop(0, NGRP)
        def _(grp):
            gbase = base + grp * IDXG
            @pl.when(gbase < NREAL_CH)
            def _():
                pltpu.sync_copy(ei_hbm.at[1, pl.ds(gbase, IDXG)], col_v)
                if gather:
                    pltpu.sync_copy(ei_hbm.at[0, pl.ds(gbase, IDXG)], row_v)

            @pl.when(gbase >= NREAL_CH)
            def _():
                pltpu.sync_copy(pad_hbm.at[pl.ds(gbase - NREAL_CH, IDXG)],
                                col_v)
                if gather:
                    pltpu.sync_copy(pad_hbm.at[pl.ds(gbase - NREAL_CH, IDXG)],
                                    row_v)

            if gather:
                for b in range(NBUF):
                    pltpu.async_copy(xs_hbm.at[row_v.at[b]], msg[b], gsem[b])

                @pl.loop(0, IDXG - NBUF, step=NBUF)
                def _(k):
                    for b in range(NBUF):
                        pltpu.make_async_copy(
                            xs_hbm.at[row_v.at[k + b]], msg[b], gsem[b]).wait()
                        pltpu.async_copy(msg[b], acc.at[col_v.at[k + b]],
                                         ssem[b], add=True)
                    for b in range(NBUF):
                        pltpu.make_async_copy(
                            msg[b], acc.at[col_v.at[k + b]], ssem[b]).wait()
                        pltpu.async_copy(
                            xs_hbm.at[row_v.at[k + NBUF + b]], msg[b], gsem[b])

                k0 = IDXG - NBUF
                for b in range(NBUF):
                    pltpu.make_async_copy(
                        xs_hbm.at[row_v.at[k0 + b]], msg[b], gsem[b]).wait()
                    pltpu.sync_copy(msg[b], acc.at[col_v.at[k0 + b]], add=True)
            else:
                @pl.loop(0, IDXG, step=NBUF)
                def _(k):
                    for b in range(NBUF):
                        pltpu.async_copy(msg[0], acc.at[col_v.at[k + b]],
                                         ssem[b], add=True)
                    for b in range(NBUF):
                        pltpu.make_async_copy(
                            msg[0], acc.at[col_v.at[k + b]], ssem[b]).wait()

        plsc.subcore_barrier()
        pltpu.sync_copy(acc.at[pl.ds(r0, ROWS_PER_SUB)],
                        out_hbm.at[c, pl.ds(r0, ROWS_PER_SUB)])

    return prop


def _tcB_body(deg_ref, x_ref, xs1_ref, dinvb_ref):
    cnt = deg_ref[0, :, 0:1] + deg_ref[1, :, 0:1] + 1.0
    dinv = jax.lax.rsqrt(cnt)
    dinvb = jnp.broadcast_to(dinv, (NP, D_IN))
    dinvb_ref[...] = dinvb
    xs1_ref[0:N, :] = dinvb[0:N, :] * x_ref[...]
    xs1_ref[N:NP, :] = jnp.zeros((NP - N, D_IN), jnp.float32)


def _tcB(deg_part, x):
    return pl.pallas_call(
        _tcB_body,
        out_shape=(
            jax.ShapeDtypeStruct((NP, D_IN), jnp.float32),
            jax.ShapeDtypeStruct((NP, D_IN), jnp.float32),
        ),
    )(deg_part, x)


def _tcD_body(s_ref, xs1_ref, dv_ref, W1_ref, b1_ref, W2_ref, xs2_ref):
    i = pl.program_id(0)
    dv = dv_ref[...]
    y1 = dv * (s_ref[0] + s_ref[1] + xs1_ref[...])
    h1 = jnp.maximum(jnp.dot(y1, W1_ref[...],
                             preferred_element_type=jnp.float32) + b1_ref[...], 0.0)
    t = jnp.dot(h1, W2_ref[...], preferred_element_type=jnp.float32)
    rowid = i * R + jax.lax.broadcasted_iota(jnp.int32, (R, 1), 0)
    xs2_ref[...] = jnp.where(rowid < N, dv[:, :D_OUT] * t, 0.0)


def _tcD(s1, xs1, dinvb, W1, b1, W2):
    return pl.pallas_call(
        _tcD_body,
        grid=(NBLK,),
        in_specs=[
            pl.BlockSpec((NC, R, D_IN), lambda i: (0, i, 0)),
            pl.BlockSpec((R, D_IN), lambda i: (i, 0)),
            pl.BlockSpec((R, D_IN), lambda i: (i, 0)),
            pl.BlockSpec((D_IN, D_HID), lambda i: (0, 0)),
            pl.BlockSpec((1, D_HID), lambda i: (0, 0)),
            pl.BlockSpec((D_HID, D_OUT), lambda i: (0, 0)),
        ],
        out_specs=pl.BlockSpec((R, D_OUT), lambda i: (i, 0)),
        out_shape=jax.ShapeDtypeStruct((NP, D_OUT), jnp.float32),
    )(s1, xs1, dinvb, W1, b1.reshape(1, D_HID), W2)


def _tcF_body(s2_ref, xs2_ref, dv_ref, b2_ref, batch_ref, Wf_ref, bf_ref,
              out_ref, S_acc):
    i = pl.program_id(0)

    @pl.when(i == 0)
    def _():
        S_acc[...] = jnp.zeros((G, 2 * D_OUT), jnp.float32)

    dv = dv_ref[:, :D_OUT]
    h2 = jnp.maximum(
        dv * (s2_ref[0] + s2_ref[1] + xs2_ref[...]) + b2_ref[...], 0.0)
    b = batch_ref[0, 0, :]
    onehot = (b[:, None] == jax.lax.broadcasted_iota(
        jnp.int32, (1, G), 1)).astype(jnp.float32)
    hext = jnp.concatenate([h2, jnp.ones((R, D_OUT), jnp.float32)], axis=1)
    S_acc[...] += jax.lax.dot_general(
        onehot, hext, (((0,), (0,)), ((), ())),
        preferred_element_type=jnp.float32)

    @pl.when(i == NBLK - 1)
    def _():
        S = S_acc[...]
        pooled = S[:, :D_OUT] / jnp.maximum(S[:, D_OUT:D_OUT + 1], 1.0)
        out_ref[...] = jnp.dot(pooled, Wf_ref[...],
                               preferred_element_type=jnp.float32) + bf_ref[...]


def _tcF(s2, xs2, dinvb, b2, batch3d, Wf, bf):
    return pl.pallas_call(
        _tcF_body,
        grid=(NBLK,),
        in_specs=[
            pl.BlockSpec((NC, R, D_OUT), lambda i: (0, i, 0)),
            pl.BlockSpec((R, D_OUT), lambda i: (i, 0)),
            pl.BlockSpec((R, D_IN), lambda i: (i, 0)),
            pl.BlockSpec((1, D_OUT), lambda i: (0, 0)),
            pl.BlockSpec((1, 1, R), lambda i: (i, 0, 0)),
            pl.BlockSpec((D_OUT, N_CLASSES), lambda i: (0, 0)),
            pl.BlockSpec((1, N_CLASSES), lambda i: (0, 0)),
        ],
        out_specs=pl.BlockSpec((G, N_CLASSES), lambda i: (0, 0)),
        out_shape=jax.ShapeDtypeStruct((G, N_CLASSES), jnp.float32),
        scratch_shapes=[pltpu.VMEM((G, 2 * D_OUT), jnp.float32)],
    )(s2, xs2, dinvb, b2.reshape(1, D_OUT), batch3d, Wf, bf.reshape(1, N_CLASSES))


def kernel(x, edge_index, batch, W1, b1, W2, b2, Wf, bf):
    ei3 = edge_index.reshape(2, NREAL_CH, CHUNK)
    pad2 = (N + (jnp.arange(PAD_CH * CHUNK, dtype=jnp.int32) % PADROWS)
            ).reshape(PAD_CH, CHUNK)
    ones16 = jnp.ones((CHUNK, 16), jnp.float32)
    z16 = jnp.zeros((NP, 16), jnp.float32)
    z128 = jnp.zeros((NP, D_IN), jnp.float32)
    z64 = jnp.zeros((NP, D_OUT), jnp.float32)
    batch3d = jnp.pad(batch, (0, NP - N), constant_values=G).reshape(NBLK, 1, R)

    deg_part = _make_propagate(16, gather=False)(
        ones16, ei3, pad2, z16)
    xs1, dinvb = _tcB(deg_part, x)
    s1 = _make_propagate(128)(xs1, ei3, pad2, z128)
    xs2 = _tcD(s1, xs1, dinvb, W1, b1, W2)
    s2 = _make_propagate(64)(xs2, ei3, pad2, z64)
    return _tcF(s2, xs2, dinvb, b2, batch3d, Wf, bf)


# speedup vs baseline: 1.0670x; 1.0509x over previous
import functools

import jax
import jax.numpy as jnp
from jax.experimental import pallas as pl
from jax.experimental.pallas import tpu as pltpu
from jax.experimental.pallas import tpu_sc as plsc

N = 10000
E = 320000
G = 64
D_IN = 128
D_HID = 512
D_OUT = 64
N_CLASSES = 2

NC = 2
NS = 16
CHUNK = 64
NP = 10752
PADROWS = 512
EPAD = 327680
NCHUNKS = EPAD // CHUNK
NREAL_CH = E // CHUNK
PAD_CH = (EPAD - E) // CHUNK
CH_PER_CORE = NCHUNKS // NC
CH_PER_SUB = CH_PER_CORE // NS
ROWS_PER_SUB = NP // NS

R = 1344
NBLK = NP // R


@functools.lru_cache(maxsize=None)
def _make_propagate(D, gather=True):
    mesh = plsc.VectorSubcoreMesh(core_axis_name="c", subcore_axis_name="s")
    NBUF = 4
    IDXG = 40
    csub = EPAD // (NC * NS * CHUNK)
    ccore = csub * NS
    NGRP = csub // IDXG

    @functools.partial(
        pl.kernel,
        mesh=mesh,
        out_type=jax.ShapeDtypeStruct((NC, NP, 128), jnp.float32),
        scratch_types=[
            pltpu.VMEM_SHARED((NP, D), jnp.float32),
            pltpu.VMEM((IDXG, CHUNK), jnp.int32),
            pltpu.VMEM((IDXG, CHUNK), jnp.int32),
        ]
        + [pltpu.VMEM((CHUNK, D), jnp.float32) for _ in range(NBUF)]
        + [pltpu.SemaphoreType.DMA for _ in range(2 * NBUF)],
        compiler_params=pltpu.CompilerParams(use_tc_tiling_on_sc=False),
    )
    def prop(xs_hbm, ei_hbm, pad_hbm, zeros_hbm, out_hbm, acc, row_v, col_v,
             *bufs_and_sems):
        msg = bufs_and_sems[:NBUF]
        gsem = bufs_and_sems[NBUF:2 * NBUF]
        ssem = bufs_and_sems[2 * NBUF:]
        c = jax.lax.axis_index("c")
        s = jax.lax.axis_index("s")
        r0 = s * ROWS_PER_SUB
        base = c * ccore + s * csub
        pltpu.sync_copy(zeros_hbm.at[pl.ds(r0, ROWS_PER_SUB)],
                        acc.at[pl.ds(r0, ROWS_PER_SUB)])
        if not gather:
            pltpu.sync_copy(xs_hbm, msg[0])
        plsc.subcore_barrier()

        @pl.loop(0, NGRP)
        def _(grp):
            gbase = base + grp * IDXG
            @pl.when(gbase < NREAL_CH)
            def _():
                pltpu.sync_copy(ei_hbm.at[1, pl.ds(gbase, IDXG)], col_v)
                if gather:
                    pltpu.sync_copy(ei_hbm.at[0, pl.ds(gbase, IDXG)], row_v)

            @pl.when(gbase >= NREAL_CH)
            def _():
                pltpu.sync_copy(pad_hbm.at[pl.ds(gbase - NREAL_CH, IDXG)],
                                col_v)
                if gather:
                    pltpu.sync_copy(pad_hbm.at[pl.ds(gbase - NREAL_CH, IDXG)],
                                    row_v)

            if gather:
                for b in range(NBUF):
                    pltpu.async_copy(xs_hbm.at[row_v.at[b]], msg[b], gsem[b])

                @pl.loop(0, IDXG - NBUF, step=NBUF)
                def _(k):
                    for b in range(NBUF):
                        pltpu.make_async_copy(
                            xs_hbm.at[row_v.at[k + b]], msg[b], gsem[b]).wait()
                        pltpu.async_copy(msg[b], acc.at[col_v.at[k + b]],
                                         ssem[b], add=True)
                    for b in range(NBUF):
                        pltpu.make_async_copy(
                            msg[b], acc.at[col_v.at[k + b]], ssem[b]).wait()
                        pltpu.async_copy(
                            xs_hbm.at[row_v.at[k + NBUF + b]], msg[b], gsem[b])

                k0 = IDXG - NBUF
                for b in range(NBUF):
                    pltpu.make_async_copy(
                        xs_hbm.at[row_v.at[k0 + b]], msg[b], gsem[b]).wait()
                    pltpu.sync_copy(msg[b], acc.at[col_v.at[k0 + b]], add=True)
            else:
                @pl.loop(0, IDXG, step=NBUF)
                def _(k):
                    for b in range(NBUF):
                        pltpu.async_copy(msg[0], acc.at[col_v.at[k + b]],
                                         ssem[b], add=True)
                    for b in range(NBUF):
                        pltpu.make_async_copy(
                            msg[0], acc.at[col_v.at[k + b]], ssem[b]).wait()

        plsc.subcore_barrier()
        pltpu.sync_copy(acc.at[pl.ds(r0, ROWS_PER_SUB)],
                        out_hbm.at[c, pl.ds(r0, ROWS_PER_SUB), pl.ds(0, D)])

    return prop


def _tcB_body(deg_ref, x_ref, xs1_ref, dinvb_ref):
    cnt = deg_ref[0, :, 0:1] + deg_ref[1, :, 0:1] + 1.0
    dinv = jax.lax.rsqrt(cnt)
    dinvb = jnp.broadcast_to(dinv, (NP, D_IN))
    dinvb_ref[...] = dinvb
    xs1_ref[0:N, :] = dinvb[0:N, :] * x_ref[...]
    xs1_ref[N:NP, :] = jnp.zeros((NP - N, D_IN), jnp.float32)


def _tcB(deg_part, x):
    return pl.pallas_call(
        _tcB_body,
        out_shape=(
            jax.ShapeDtypeStruct((NP, D_IN), jnp.float32),
            jax.ShapeDtypeStruct((NP, D_IN), jnp.float32),
        ),
    )(deg_part, x)


def _tcD_body(s_ref, xs1_ref, dv_ref, W1_ref, b1_ref, W2_ref, xs2_ref):
    i = pl.program_id(0)
    dv = dv_ref[...]
    y1 = dv * (s_ref[0] + s_ref[1] + xs1_ref[...])
    h1 = jnp.maximum(jnp.dot(y1, W1_ref[...],
                             preferred_element_type=jnp.float32) + b1_ref[...], 0.0)
    t = jnp.dot(h1, W2_ref[...], preferred_element_type=jnp.float32)
    rowid = i * R + jax.lax.broadcasted_iota(jnp.int32, (R, 1), 0)
    xs2_ref[...] = jnp.where(rowid < N, dv[:, :D_OUT] * t, 0.0)


def _tcD(s1, xs1, dinvb, W1, b1, W2):
    return pl.pallas_call(
        _tcD_body,
        grid=(NBLK,),
        in_specs=[
            pl.BlockSpec((NC, R, D_IN), lambda i: (0, i, 0)),
            pl.BlockSpec((R, D_IN), lambda i: (i, 0)),
            pl.BlockSpec((R, D_IN), lambda i: (i, 0)),
            pl.BlockSpec((D_IN, D_HID), lambda i: (0, 0)),
            pl.BlockSpec((1, D_HID), lambda i: (0, 0)),
            pl.BlockSpec((D_HID, D_OUT), lambda i: (0, 0)),
        ],
        out_specs=pl.BlockSpec((R, D_OUT), lambda i: (i, 0)),
        out_shape=jax.ShapeDtypeStruct((NP, D_OUT), jnp.float32),
    )(s1, xs1, dinvb, W1, b1.reshape(1, D_HID), W2)


def _tcF_body(s2_ref, xs2_ref, dv_ref, b2_ref, batch_ref, Wf_ref, bf_ref,
              out_ref, S_acc):
    i = pl.program_id(0)

    @pl.when(i == 0)
    def _():
        S_acc[...] = jnp.zeros((G, 2 * D_OUT), jnp.float32)

    dv = dv_ref[:, :D_OUT]
    s2sum = s2_ref[0, :, 0:D_OUT] + s2_ref[1, :, 0:D_OUT]
    h2 = jnp.maximum(dv * (s2sum + xs2_ref[...]) + b2_ref[...], 0.0)
    b = batch_ref[0, 0, :]
    onehot = (b[:, None] == jax.lax.broadcasted_iota(
        jnp.int32, (1, G), 1)).astype(jnp.float32)
    hext = jnp.concatenate([h2, jnp.ones((R, D_OUT), jnp.float32)], axis=1)
    S_acc[...] += jax.lax.dot_general(
        onehot, hext, (((0,), (0,)), ((), ())),
        preferred_element_type=jnp.float32)

    @pl.when(i == NBLK - 1)
    def _():
        S = S_acc[...]
        pooled = S[:, :D_OUT] / jnp.maximum(S[:, D_OUT:D_OUT + 1], 1.0)
        out_ref[...] = jnp.dot(pooled, Wf_ref[...],
                               preferred_element_type=jnp.float32) + bf_ref[...]


def _tcF(s2, xs2, dinvb, b2, batch3d, Wf, bf):
    return pl.pallas_call(
        _tcF_body,
        grid=(NBLK,),
        in_specs=[
            pl.BlockSpec((NC, R, 128), lambda i: (0, i, 0)),
            pl.BlockSpec((R, D_OUT), lambda i: (i, 0)),
            pl.BlockSpec((R, D_IN), lambda i: (i, 0)),
            pl.BlockSpec((1, D_OUT), lambda i: (0, 0)),
            pl.BlockSpec((1, 1, R), lambda i: (i, 0, 0)),
            pl.BlockSpec((D_OUT, N_CLASSES), lambda i: (0, 0)),
            pl.BlockSpec((1, N_CLASSES), lambda i: (0, 0)),
        ],
        out_specs=pl.BlockSpec((G, N_CLASSES), lambda i: (0, 0)),
        out_shape=jax.ShapeDtypeStruct((G, N_CLASSES), jnp.float32),
        scratch_shapes=[pltpu.VMEM((G, 2 * D_OUT), jnp.float32)],
    )(s2, xs2, dinvb, b2.reshape(1, D_OUT), batch3d, Wf, bf.reshape(1, N_CLASSES))


def kernel(x, edge_index, batch, W1, b1, W2, b2, Wf, bf):
    ei3 = edge_index.reshape(2, NREAL_CH, CHUNK)
    pad2 = (N + (jnp.arange(PAD_CH * CHUNK, dtype=jnp.int32) % PADROWS)
            ).reshape(PAD_CH, CHUNK)
    ones16 = jnp.ones((CHUNK, 16), jnp.float32)
    z16 = jnp.zeros((NP, 16), jnp.float32)
    z128 = jnp.zeros((NP, D_IN), jnp.float32)
    z64 = jnp.zeros((NP, D_OUT), jnp.float32)
    batch3d = jnp.pad(batch, (0, NP - N), constant_values=G).reshape(NBLK, 1, R)

    deg_part = _make_propagate(16, gather=False)(
        ones16, ei3, pad2, z16)
    xs1, dinvb = _tcB(deg_part, x)
    s1 = _make_propagate(128)(xs1, ei3, pad2, z128)
    xs2 = _tcD(s1, xs1, dinvb, W1, b1, W2)
    s2 = _make_propagate(64)(xs2, ei3, pad2, z64)
    return _tcF(s2, xs2, dinvb, b2, batch3d, Wf, bf)
